# baseline (speedup 1.0000x reference)
import functools

import jax
import jax.numpy as jnp
from jax import lax
from jax.experimental import pallas as pl
from jax.experimental.pallas import tpu as pltpu
from jax.experimental.pallas import tpu_sc as plsc

B, S, NC, NN, V, D = 1024, 20, 26, 13, 100000, 32
NF = NC + NN
EPS = 1e-12

NW = 32
NPAIR = S * NC
MAXK = (NPAIR + NW - 1) // NW

G = 128 // D
B4 = B // G
Bb4 = 128


def _sc_gather(xq, tbl):
  mesh = plsc.VectorSubcoreMesh(core_axis_name="c", subcore_axis_name="s")

  @functools.partial(
      pl.kernel,
      mesh=mesh,
      out_type=jax.ShapeDtypeStruct((S, NC, 8, 128, D), jnp.float32),
      compiler_params=pltpu.CompilerParams(use_tc_tiling_on_sc=False),
      scratch_types=[
          pltpu.VMEM((2, 8, 128), jnp.int32),
          pltpu.VMEM((2, 8, 128, D), jnp.float32),
          pltpu.SemaphoreType.DMA,
          pltpu.SemaphoreType.DMA,
      ],
  )
  def k(xq_hbm, tbl_hbm, gt_hbm, idx_v, vals_v, sem, dsem):
    wid = lax.axis_index("s") * 2 + lax.axis_index("c")

    def pair(i, q):
      p = wid + NW * i
      idx_q = idx_v.at[q]
      vals_q = vals_v.at[q]

      @pl.when(p < NPAIR)
      def _():
        s = p // NC
        c = p - s * NC

        @pl.when(i >= 2)
        def _():
          pltpu.make_async_copy(vals_q, gt_hbm.at[0, 0], dsem).wait()

        pltpu.sync_copy(xq_hbm.at[c, s], idx_q)

        for j in range(8):
          pltpu.async_copy(
              tbl_hbm.at[c].at[idx_q.at[j]], vals_q.at[j], sem)
        for j in range(8):
          pltpu.make_async_copy(gt_hbm.at[0, 0, 0], vals_q.at[j], sem).wait()
        pltpu.async_copy(vals_q, gt_hbm.at[s, c], dsem)

    def body(kk, carry):
      pair(2 * kk, 0)
      pair(2 * kk + 1, 1)
      return carry

    lax.fori_loop(0, (MAXK + 1) // 2, body, None)
    pltpu.make_async_copy(vals_v.at[0], gt_hbm.at[0, 0], dsem).wait()
    pltpu.make_async_copy(vals_v.at[1], gt_hbm.at[0, 0], dsem).wait()

  return k(xq, tbl)


def _ln_body(gt_ref, xn_ref, p4_ref, nw_ref, mseg_ref, w_ref, b_ref, out_ref):
  g = gt_ref[0]
  xn4 = xn_ref[0]
  p4 = p4_ref[...]
  mseg = mseg_ref[...]
  xb = lax.dot_general(
      xn4.reshape(NN * Bb4, G), p4,
      (((1,), (0,)), ((), ())),
      preferred_element_type=jnp.float32,
      precision=lax.Precision.HIGHEST).reshape(NN, Bb4, 128)
  xe = xb * nw_ref[...][:, None, :]
  x = jnp.concatenate([g, xe], axis=0).reshape(NF * Bb4, 128)
  s1 = lax.dot_general(x, mseg, (((1,), (0,)), ((), ())),
                       preferred_element_type=jnp.float32,
                       precision=lax.Precision.HIGHEST)
  mu = s1 * (1.0 / D)
  t = x - mu
  s2 = lax.dot_general(t * t, mseg, (((1,), (0,)), ((), ())),
                       preferred_element_type=jnp.float32,
                       precision=lax.Precision.HIGHEST)
  xh = t * lax.rsqrt(s2 * (1.0 / D) + EPS)
  out = xh * w_ref[...] + b_ref[...]
  out_ref[0] = out.reshape(NF, Bb4, 128)


def _tc_ln(gtv, xn4, p4, nw128, mseg, w128, b128):
  grid = (S, B4 // Bb4)
  return pl.pallas_call(
      _ln_body,
      grid=grid,
      in_specs=[
          pl.BlockSpec((1, NC, Bb4, 128), lambda i, j: (i, 0, j, 0)),
          pl.BlockSpec((1, NN, Bb4, G), lambda i, j: (i, 0, j, 0)),
          pl.BlockSpec((G, 128), lambda i, j: (0, 0)),
          pl.BlockSpec((NN, 128), lambda i, j: (0, 0)),
          pl.BlockSpec((128, 128), lambda i, j: (0, 0)),
          pl.BlockSpec((1, 128), lambda i, j: (0, 0)),
          pl.BlockSpec((1, 128), lambda i, j: (0, 0)),
      ],
      out_specs=pl.BlockSpec((1, NF, Bb4, 128), lambda i, j: (i, 0, j, 0)),
      out_shape=jax.ShapeDtypeStruct((S, NF, B4, 128), jnp.float32),
      compiler_params=pltpu.CompilerParams(
          dimension_semantics=("arbitrary", "arbitrary")),
  )(gtv, xn4, p4, nw128, mseg, w128, b128)


def kernel(xc, xn, cls_tables, num_weight, ln_weight, ln_bias):
  xq = jnp.transpose(xc, (2, 1, 0)).reshape(NC, S, 8, 128)
  gt = _sc_gather(xq, cls_tables)
  gtv = gt.reshape(S, NC, B4, 128)
  xn4 = jnp.transpose(xn, (1, 2, 0)).reshape(S, NN, B4, G)
  lane = jnp.arange(128, dtype=jnp.int32)
  p4 = (lane[None, :] // D == jnp.arange(G, dtype=jnp.int32)[:, None]
        ).astype(jnp.float32)
  mseg = (lane[:, None] // D == lane[None, :] // D
          ).astype(jnp.float32)
  nw128 = jnp.tile(num_weight, (1, G))
  w128 = jnp.tile(ln_weight, G).reshape(1, 128)
  b128 = jnp.tile(ln_bias, G).reshape(1, 128)
  o = _tc_ln(gtv, xn4, p4, nw128, mseg, w128, b128)
  o4 = o.reshape(S, NF, B, D)
  return jnp.transpose(o4, (2, 0, 1, 3))


# speedup vs baseline: 1.3626x; 1.3626x over previous
import functools

import jax
import jax.numpy as jnp
from jax import lax
from jax.experimental import pallas as pl
from jax.experimental.pallas import tpu as pltpu
from jax.experimental.pallas import tpu_sc as plsc

B, S, NC, NN, V, D = 1024, 20, 26, 13, 100000, 32
NF = NC + NN
EPS = 1e-12

NW = 32
NPAIR = S * NC
MAXK = (NPAIR + NW - 1) // NW


def _sc_gather(xq, tt):
  mesh = plsc.VectorSubcoreMesh(core_axis_name="c", subcore_axis_name="s")

  @functools.partial(
      pl.kernel,
      mesh=mesh,
      out_type=jax.ShapeDtypeStruct((S, NC, D, 8, 128), jnp.float32),
      compiler_params=pltpu.CompilerParams(use_tc_tiling_on_sc=False),
      scratch_types=[
          pltpu.VMEM((2, 8, 128), jnp.int32),
          pltpu.VMEM((2, D, 8, 128), jnp.float32),
          pltpu.SemaphoreType.DMA,
          pltpu.SemaphoreType.DMA,
      ],
  )
  def k(xq_hbm, tt_hbm, gt_hbm, idx_v, vals_v, sem, dsem):
    wid = lax.axis_index("s") * 2 + lax.axis_index("c")

    def pair(i, q):
      p = wid + NW * i
      idx_q = idx_v.at[q]
      vals_q = vals_v.at[q]

      @pl.when(p < NPAIR)
      def _():
        s = p // NC
        c = p - s * NC

        @pl.when(i >= 2)
        def _():
          pltpu.make_async_copy(vals_q, gt_hbm.at[0, 0], dsem).wait()

        pltpu.sync_copy(xq_hbm.at[c, s], idx_q)

        def dbody(d, carry):
          for j in range(8):
            pltpu.async_copy(
                tt_hbm.at[c, d].at[idx_q.at[j]], vals_q.at[d, j], sem)
          @pl.when(d >= 4)
          def _():
            pltpu.make_async_copy(gt_hbm.at[0, 0, 0], vals_q.at[d - 4], sem
                                  ).wait()
          return carry

        lax.fori_loop(0, D, dbody, None)
        for dd in (D - 4, D - 3, D - 2, D - 1):
          pltpu.make_async_copy(gt_hbm.at[0, 0, 0], vals_q.at[dd], sem).wait()
        pltpu.async_copy(vals_q, gt_hbm.at[s, c], dsem)

    def body(kk, carry):
      pair(2 * kk, 0)
      pair(2 * kk + 1, 1)
      return carry

    lax.fori_loop(0, (MAXK + 1) // 2, body, None)
    pltpu.make_async_copy(vals_v.at[0], gt_hbm.at[0, 0], dsem).wait()
    pltpu.make_async_copy(vals_v.at[1], gt_hbm.at[0, 0], dsem).wait()

  return k(xq, tt)


def _ln_body(gt_ref, xn_ref, nw_ref, w_ref, b_ref, out_ref):
  g = gt_ref[0]
  xnv = xn_ref[0]
  nw = nw_ref[...]
  xe = xnv[:, None, :] * nw[:, :, None]
  x = jnp.concatenate([g, xe], axis=0)
  u = jnp.mean(x, axis=1, keepdims=True)
  t = x - u
  s = jnp.mean(t * t, axis=1, keepdims=True)
  xh = t * lax.rsqrt(s + EPS)
  out_ref[0] = w_ref[...][None, :, :] * xh + b_ref[...][None, :, :]


def _tc_ln(gt4, xnT, nw, w2, b2):
  Bb = 512
  grid = (S, B // Bb)
  return pl.pallas_call(
      _ln_body,
      grid=grid,
      in_specs=[
          pl.BlockSpec((1, NC, D, Bb), lambda i, j: (i, 0, 0, j)),
          pl.BlockSpec((1, NN, Bb), lambda i, j: (i, 0, j)),
          pl.BlockSpec((NN, D), lambda i, j: (0, 0)),
          pl.BlockSpec((D, 1), lambda i, j: (0, 0)),
          pl.BlockSpec((D, 1), lambda i, j: (0, 0)),
      ],
      out_specs=pl.BlockSpec((1, NF, D, Bb), lambda i, j: (i, 0, 0, j)),
      out_shape=jax.ShapeDtypeStruct((S, NF, D, B), jnp.float32),
      compiler_params=pltpu.CompilerParams(
          dimension_semantics=("arbitrary", "arbitrary")),
  )(gt4, xnT, nw, w2, b2)


def kernel(xc, xn, cls_tables, num_weight, ln_weight, ln_bias):
  tt = jnp.transpose(cls_tables, (0, 2, 1))
  xq = jnp.transpose(xc, (2, 1, 0)).reshape(NC, S, 8, 128)
  gt = _sc_gather(xq, tt)
  gt4 = gt.reshape(S, NC, D, B)
  xnT = jnp.transpose(xn, (1, 2, 0))
  o = _tc_ln(gt4, xnT, num_weight,
             ln_weight.reshape(D, 1), ln_bias.reshape(D, 1))
  return jnp.transpose(o, (3, 0, 1, 2))
